# baseline (speedup 1.0000x reference)
import jax
import jax.numpy as jnp
from jax.experimental import pallas as pl


def _body(col_ref, row_ref, o_ref):
    d, w = col_ref.shape
    h = row_ref.shape[1]
    colT = col_ref[...]
    rowT = row_ref[...]
    o_ref[0, :d] = jnp.broadcast_to(colT[:, None, :], (d, h, w))
    o_ref[0, d:] = jnp.broadcast_to(rowT[:, :, None], (d, h, w))


def kernel(x, mask, row_embed, col_embed):
    b = x.shape[0]
    h, w = x.shape[-2], x.shape[-1]
    d = col_embed.shape[-1]
    colT = col_embed[:w].T
    rowT = row_embed[:h].T
    out = pl.pallas_call(
        _body,
        grid=(b,),
        in_specs=[
            pl.BlockSpec((d, w), lambda i: (0, 0)),
            pl.BlockSpec((d, h), lambda i: (0, 0)),
        ],
        out_specs=pl.BlockSpec((1, 2 * d, h, w), lambda i: (i, 0, 0, 0)),
        out_shape=jax.ShapeDtypeStruct((b, 2 * d, h, w), jnp.float32),
    )(colT, rowT)
    return out


# speedup vs baseline: 6.5117x; 6.5117x over previous
import jax
import jax.numpy as jnp
from jax.experimental import pallas as pl


def _body(col_ref, row_ref, o_ref):
    w, d = col_ref.shape
    h = row_ref.shape[0]
    col = col_ref[...]
    row = row_ref[...]
    o_ref[0, :, :, :d] = jnp.broadcast_to(col[None, :, :], (h, w, d))
    o_ref[0, :, :, d:] = jnp.broadcast_to(row[:, None, :], (h, w, d))


def kernel(x, mask, row_embed, col_embed):
    b = x.shape[0]
    h, w = x.shape[-2], x.shape[-1]
    d = col_embed.shape[-1]
    out_nat = pl.pallas_call(
        _body,
        grid=(b,),
        in_specs=[
            pl.BlockSpec((w, d), lambda i: (0, 0)),
            pl.BlockSpec((h, d), lambda i: (0, 0)),
        ],
        out_specs=pl.BlockSpec((1, h, w, 2 * d), lambda i: (i, 0, 0, 0)),
        out_shape=jax.ShapeDtypeStruct((b, h, w, 2 * d), jnp.float32),
    )(col_embed[:w], row_embed[:h])
    return jnp.transpose(out_nat, (0, 3, 1, 2))


# speedup vs baseline: 7.1114x; 1.0921x over previous
import jax
import jax.numpy as jnp
from jax.experimental import pallas as pl
from jax.experimental.pallas import tpu as pltpu


def _make_body(b):
    def _body(col_ref, row_ref, o_ref, scratch, sems):
        w, d = col_ref.shape
        h = row_ref.shape[0]
        scratch[:, :, :d] = jnp.broadcast_to(col_ref[...][None, :, :], (h, w, d))
        scratch[:, :, d:] = jnp.broadcast_to(row_ref[...][:, None, :], (h, w, d))
        copies = [
            pltpu.make_async_copy(scratch, o_ref.at[i], sems.at[i])
            for i in range(b)
        ]
        for c in copies:
            c.start()
        for c in copies:
            c.wait()

    return _body


def kernel(x, mask, row_embed, col_embed):
    b = x.shape[0]
    h, w = x.shape[-2], x.shape[-1]
    d = col_embed.shape[-1]
    out_nat = pl.pallas_call(
        _make_body(b),
        in_specs=[
            pl.BlockSpec((w, d), lambda: (0, 0)),
            pl.BlockSpec((h, d), lambda: (0, 0)),
        ],
        out_specs=pl.BlockSpec(memory_space=pl.ANY),
        out_shape=jax.ShapeDtypeStruct((b, h, w, 2 * d), jnp.float32),
        scratch_shapes=[
            pltpu.VMEM((h, w, 2 * d), jnp.float32),
            pltpu.SemaphoreType.DMA((b,)),
        ],
    )(col_embed[:w], row_embed[:h])
    return jnp.transpose(out_nat, (0, 3, 1, 2))


# speedup vs baseline: 10.1540x; 1.4279x over previous
import jax
import jax.numpy as jnp
from jax.experimental import pallas as pl
from jax.experimental.pallas import tpu as pltpu


def _make_body(b):
    def _body(col_ref, row_ref, o_ref, scratch, sems):
        w, d = col_ref.shape
        h = row_ref.shape[0]
        scratch[:, :, :d] = jnp.broadcast_to(col_ref[...][None, :, :], (h, w, d))
        scratch[:, :, d:] = jnp.broadcast_to(row_ref[...][:, None, :], (h, w, d))
        copies = [
            pltpu.make_async_copy(scratch, o_ref.at[i], sems.at[i])
            for i in range(b)
        ]
        for c in copies:
            c.start()
        for c in copies:
            c.wait()

    return _body


def kernel(x, mask, row_embed, col_embed):
    b = x.shape[0]
    h, w = x.shape[-2], x.shape[-1]
    d = col_embed.shape[-1]
    out_nat = pl.pallas_call(
        _make_body(b),
        grid=(1,),
        in_specs=[
            pl.BlockSpec((w, d), lambda i: (0, 0)),
            pl.BlockSpec((h, d), lambda i: (0, 0)),
        ],
        out_specs=pl.BlockSpec(memory_space=pl.ANY),
        out_shape=jax.ShapeDtypeStruct((b, h, w, 2 * d), jnp.float32),
        scratch_shapes=[
            pltpu.VMEM((h, w, 2 * d), jnp.float32),
            pltpu.SemaphoreType.DMA((b,)),
        ],
    )(col_embed, row_embed)
    return jnp.transpose(out_nat, (0, 3, 1, 2))
